# Initial kernel scaffold; baseline (speedup 1.0000x reference)
import jax
import jax.numpy as jnp
from jax.experimental import pallas as pl


def kernel(x, table):
    raise NotImplementedError("write your pallas kernel here")


# speedup vs baseline: 16.2664x; 16.2664x over previous
import functools

import jax
import jax.numpy as jnp
from jax import lax
from jax.experimental import pallas as pl
from jax.experimental.pallas import tpu as pltpu
from jax.experimental.pallas import tpu_sc as plsc

NB_CLASSES = 1000
EMB = 50
V = 20
NC, NS, L = 2, 16, 16
NW = NC * NS

N_ROWS = 1024 * 50
ROWS_PER_W = N_ROWS // NW
CHUNK = 800
N_CHUNKS = ROWS_PER_W // CHUNK
GROUPS = CHUNK // L


def _tree_sum(terms):
    while len(terms) > 1:
        nxt = [terms[i] + terms[i + 1] for i in range(0, len(terms) - 1, 2)]
        if len(terms) % 2:
            nxt.append(terms[-1])
        terms = nxt
    return terms[0]


def _sc_body(x_hbm, table_hbm, out_hbm, xbuf, tbuf, obuf):
    wid = lax.axis_index("s") * NC + lax.axis_index("c")
    pltpu.sync_copy(table_hbm, tbuf)
    lane = lax.iota(jnp.int32, L)

    def chunk_body(c, carry):
        base = wid * ROWS_PER_W + c * CHUNK
        pltpu.sync_copy(x_hbm.at[pl.ds(base, CHUNK), :], xbuf)

        def group_body(g, carry):
            rows = g * L + lane
            idx = [
                plsc.load_gather(xbuf, [rows, jnp.full((L,), v, jnp.int32)])
                for v in range(V)
            ]
            w = []
            for a in range(V):
                m = jnp.full((L,), 0.0, jnp.float32)
                for b in range(V):
                    m = m + jnp.where(idx[a] == idx[b], 1.0, 0.0)
                w.append(1.0 / m)
            inv_u = 1.0 / _tree_sum(list(w))
            wn = [wv * inv_u for wv in w]

            addrs0 = tuple(iv * EMB for iv in idx) + (rows * EMB,)

            @plsc.parallel_loop(0, EMB, carry=addrs0, unroll=1)
            def e_body(e, cr):
                acc = _tree_sum(
                    [wn[a] * plsc.load_gather(tbuf, [cr[a]]) for a in range(V)]
                )
                plsc.store_scatter(obuf, [cr[V]], acc)
                return tuple(a + 1 for a in cr)

            return carry

        carry = lax.fori_loop(0, GROUPS, group_body, carry)
        pltpu.sync_copy(obuf, out_hbm.at[pl.ds(base * EMB, CHUNK * EMB)])
        return carry

    lax.fori_loop(0, N_CHUNKS, chunk_body, 0)


_sc_kernel = functools.partial(
    pl.kernel,
    out_type=jax.ShapeDtypeStruct((N_ROWS * EMB,), jnp.float32),
    mesh=plsc.VectorSubcoreMesh(
        core_axis_name="c", subcore_axis_name="s", num_cores=NC, num_subcores=NS
    ),
    scratch_types=[
        pltpu.VMEM((CHUNK, V), jnp.int32),
        pltpu.VMEM((NB_CLASSES * EMB,), jnp.float32),
        pltpu.VMEM((CHUNK * EMB,), jnp.float32),
    ],
    compiler_params=pltpu.CompilerParams(
        needs_layout_passes=False, use_tc_tiling_on_sc=False
    ),
)(_sc_body)


def kernel(x, table):
    b, l, v = x.shape
    xf = x.reshape(b * l, v).astype(jnp.int32)
    out = _sc_kernel(xf, table.reshape(NB_CLASSES * EMB))
    return out.reshape(b, l, EMB)
